# baseline (speedup 1.0000x reference)
import functools
import math

import jax
import jax.numpy as jnp
import numpy as np
from jax import lax
from jax.experimental import pallas as pl
from jax.experimental.pallas import tpu as pltpu
from jax.experimental.pallas import tpu_sc as plsc

VOCAB = 100000
D_MODEL = 128
LENGTH = 2048
BATCH = 4
SEQ = 2048
SCALE = math.sqrt(float(D_MODEL))


def _positional_encoding(length, depth):
    half = depth / 2
    positions = np.arange(length)[:, np.newaxis]
    depths = np.arange(half)[np.newaxis, :] / half
    angle_rates = 1.0 / (10000.0 ** depths)
    angle_rads = positions * angle_rates
    return np.concatenate([np.sin(angle_rads), np.cos(angle_rads)], axis=-1).astype(np.float32)


_INFO = plsc.get_sparse_core_info()
_NC = _INFO.num_cores
_NS = _INFO.num_subcores
_NW = _NC * _NS
_PPW = SEQ // _NW
_LANES = 16
_VPR = D_MODEL // _LANES

_POS = jnp.asarray(
    _positional_encoding(LENGTH, D_MODEL).reshape(_NW, _PPW, D_MODEL)
)


@functools.partial(
    pl.kernel,
    mesh=plsc.VectorSubcoreMesh(core_axis_name="c", subcore_axis_name="s"),
    out_type=jax.ShapeDtypeStruct((BATCH * SEQ, D_MODEL), jnp.float32),
    scratch_types=[
        pltpu.VMEM((BATCH, _PPW), jnp.int32),
        pltpu.VMEM((BATCH, _PPW, D_MODEL), jnp.float32),
        pltpu.VMEM((_PPW, D_MODEL), jnp.float32),
        pltpu.SemaphoreType.DMA,
        pltpu.SemaphoreType.DMA,
        pltpu.SemaphoreType.DMA,
        pltpu.SemaphoreType.DMA,
        pltpu.SemaphoreType.DMA,
        pltpu.SemaphoreType.DMA,
        pltpu.SemaphoreType.DMA,
    ],
)
def _emb_kernel(x_hbm, table_hbm, pos_hbm, out_hbm, idx_v, rows_v, pos_v,
                sem_idx, sem_pos, sem_out, sg0, sg1, sg2, sg3):
    wid = lax.axis_index("s") * _NC + lax.axis_index("c")
    gsems = [sg0, sg1, sg2, sg3]
    idx_cps = [
        pltpu.async_copy(x_hbm.at[b, wid], idx_v.at[b], sem_idx)
        for b in range(BATCH)
    ]
    for cp in idx_cps:
        cp.wait()
    gcps = [pltpu.async_copy(table_hbm.at[idx_v.at[0]], rows_v.at[0], gsems[0])]
    pos_cp = pltpu.async_copy(pos_hbm.at[wid], pos_v, sem_pos)
    for b in range(1, BATCH):
        gcps.append(
            pltpu.async_copy(table_hbm.at[idx_v.at[b]], rows_v.at[b], gsems[b])
        )

    def make_body(b):
        def row_body(i, _):
            for c in range(_VPR):
                sl = pl.ds(c * _LANES, _LANES)
                rows_v[b, i, sl] = rows_v[b, i, sl] * SCALE + pos_v[i, sl]
            return _
        return row_body

    pos_cp.wait()
    out_cps = []
    for b in range(BATCH):
        gcps[b].wait()
        lax.fori_loop(0, _PPW, make_body(b), 0)
        out_cps.append(
            pltpu.async_copy(
                rows_v.at[b],
                out_hbm.at[pl.ds(b * SEQ + wid * _PPW, _PPW)],
                sem_out,
            )
        )
    for cp in out_cps:
        cp.wait()


def kernel(x, table):
    xf = jnp.reshape(x, (BATCH, _NW, _PPW)).astype(jnp.int32)
    out = _emb_kernel(xf, table, _POS)
    return jnp.reshape(out, (BATCH, SEQ, D_MODEL))


# speedup vs baseline: 1.0328x; 1.0328x over previous
import functools
import math

import jax
import jax.numpy as jnp
import numpy as np
from jax import lax
from jax.experimental import pallas as pl
from jax.experimental.pallas import tpu as pltpu
from jax.experimental.pallas import tpu_sc as plsc

VOCAB = 100000
D_MODEL = 128
LENGTH = 2048
BATCH = 4
SEQ = 2048
SCALE = math.sqrt(float(D_MODEL))


def _positional_encoding(length, depth):
    half = depth / 2
    positions = np.arange(length)[:, np.newaxis]
    depths = np.arange(half)[np.newaxis, :] / half
    angle_rates = 1.0 / (10000.0 ** depths)
    angle_rads = positions * angle_rates
    return np.concatenate([np.sin(angle_rads), np.cos(angle_rads)], axis=-1).astype(np.float32)


_INFO = plsc.get_sparse_core_info()
_NC = _INFO.num_cores
_NS = _INFO.num_subcores
_NW = _NC * _NS
_PPW = SEQ // _NW
_LANES = 16
_VPR = D_MODEL // _LANES

_POS = jnp.asarray(
    _positional_encoding(LENGTH, D_MODEL).reshape(_NW, _PPW, D_MODEL)
)


@functools.partial(
    pl.kernel,
    mesh=plsc.VectorSubcoreMesh(core_axis_name="c", subcore_axis_name="s"),
    out_type=jax.ShapeDtypeStruct((BATCH, SEQ, D_MODEL), jnp.float32),
    scratch_types=[
        pltpu.VMEM((BATCH, _PPW), jnp.int32),
        pltpu.VMEM((BATCH, _PPW, D_MODEL), jnp.float32),
        pltpu.VMEM((_PPW, D_MODEL), jnp.float32),
        pltpu.SemaphoreType.DMA,
        pltpu.SemaphoreType.DMA,
        pltpu.SemaphoreType.DMA,
        pltpu.SemaphoreType.DMA,
        pltpu.SemaphoreType.DMA,
        pltpu.SemaphoreType.DMA,
        pltpu.SemaphoreType.DMA,
    ],
)
def _emb_kernel(x_hbm, table_hbm, pos_hbm, out_hbm, idx_v, rows_v, pos_v,
                sem_idx, sem_pos, sem_out, sg0, sg1, sg2, sg3):
    wid = lax.axis_index("s") * _NC + lax.axis_index("c")
    gsems = [sg0, sg1, sg2, sg3]
    idx_cps = [
        pltpu.async_copy(x_hbm.at[b, pl.ds(wid * _PPW, _PPW)], idx_v.at[b], sem_idx)
        for b in range(BATCH)
    ]
    for cp in idx_cps:
        cp.wait()
    gcps = [pltpu.async_copy(table_hbm.at[idx_v.at[0]], rows_v.at[0], gsems[0])]
    pos_cp = pltpu.async_copy(pos_hbm.at[wid], pos_v, sem_pos)
    for b in range(1, BATCH):
        gcps.append(
            pltpu.async_copy(table_hbm.at[idx_v.at[b]], rows_v.at[b], gsems[b])
        )

    def make_body(b):
        def row_body(i, _):
            for c in range(_VPR):
                sl = pl.ds(c * _LANES, _LANES)
                rows_v[b, i, sl] = rows_v[b, i, sl] * SCALE + pos_v[i, sl]
            return _
        return row_body

    pos_cp.wait()
    out_cps = []
    for b in range(BATCH):
        gcps[b].wait()
        lax.fori_loop(0, _PPW, make_body(b), 0)
        out_cps.append(
            pltpu.async_copy(
                rows_v.at[b],
                out_hbm.at[b, pl.ds(wid * _PPW, _PPW)],
                sem_out,
            )
        )
    for cp in out_cps:
        cp.wait()


def kernel(x, table):
    return _emb_kernel(x.astype(jnp.int32), table, _POS)
